# baseline (speedup 1.0000x reference)
import functools

import numpy as np

import jax
import jax.numpy as jnp
from jax import lax
from jax.experimental import pallas as pl
from jax.experimental.pallas import tpu as pltpu
from jax.experimental.pallas import tpu_sc as plsc

_N = 10000
_NP = 10240
_E = 160000
_D = 300
_DH = 150
_DP = 160
_CH = 80
_NS = 16
_RPT = _NP // _NS
_HN = _NP // 2
_AR = _HN + 512
_NCHUNK = 2048
_EP = _NCHUNK * _CH
_CPT = _NCHUNK // _NS
_CPW = _NCHUNK // (2 * _NS)
_CAP = _EP // 32
_CAPC = _CAP // _CH
_NB = 3
_D4 = 8
_NEG = -1e30

_R = 1024
_GRID = _NP // _R

_sc_mesh = plsc.VectorSubcoreMesh(core_axis_name="c", subcore_axis_name="s")

_QIDX = np.zeros(_DP, np.int32)
for _g in range(_DP // 32):
    for _i in range(16):
        _QIDX[32 * _g + _i] = 32 * _g + 2 * _i
        _QIDX[32 * _g + 16 + _i] = 32 * _g + 2 * _i + 1


def _pipelined_chunks(nch, x_hbm, dummy_f32, sball, dball, rows_in, rows_out,
                      acc, gsem, ssem, convert=None):
    nch = jnp.int32(nch)

    def _gwait(s):
        pltpu.make_async_copy(x_hbm.at[pl.ds(0, _CH)], rows_in.at[s],
                              gsem.at[s]).wait()

    def _swait(s):
        pltpu.make_async_copy(dummy_f32, rows_out.at[s], ssem.at[s]).wait()

    pltpu.async_copy(x_hbm.at[sball.at[pl.ds(0, _CH)]], rows_in.at[0],
                     gsem.at[0])

    def _chunk(g, _):
        s_g = lax.rem(g, _NB)
        nxt = g + 1

        @pl.when(nxt < nch)
        def _():
            pltpu.async_copy(x_hbm.at[sball.at[pl.ds(nxt * _CH, _CH)]],
                             rows_in.at[lax.rem(nxt, _NB)],
                             gsem.at[lax.rem(nxt, _NB)])

        _gwait(s_g)

        @pl.when(g >= _NB)
        def _():
            _swait(s_g)
        if convert is not None:
            convert(s_g)
        pltpu.async_copy(rows_out.at[s_g], acc.at[dball.at[g]], ssem.at[s_g],
                         add=True)
        return 0
    lax.fori_loop(0, nch, _chunk, 0)
    for j in range(_NB):
        @pl.when(j < nch)
        def _():
            _swait(j)


def _load_dball(row_slice_fn, dball, nch, sem, dummy):
    def _row(r, _):
        pltpu.async_copy(row_slice_fn(r), dball.at[r], sem.at[0])
        return 0
    lax.fori_loop(0, nch, _row, 0)

    def _drain(r, _):
        pltpu.make_async_copy(dummy, dball.at[0], sem.at[0]).wait()
        return 0
    lax.fori_loop(0, nch, _drain, 0)


def _agg_half_body(sid, x_hbm, psrc, pdst, counts, zeros_hbm, out_hbm, sball,
                   dball, cnts, rows_bf, rows_f, acc, gsem, ssem):
    pltpu.sync_copy(counts, cnts)

    def _convert(s):
        def _row(e, _):
            for g2 in range(_DP // 32):
                vb = plsc.bitcast(rows_bf[s, e, pl.ds(32 * g2, 32)],
                                  jnp.int32)
                rows_f[s, e, pl.ds(32 * g2, 16)] = plsc.bitcast(
                    vb << 16, jnp.float32)
                rows_f[s, e, pl.ds(32 * g2 + 16, 16)] = plsc.bitcast(
                    vb & jnp.int32(-65536), jnp.float32)
            return 0
        lax.fori_loop(0, _CH, _row, 0)

    for h in (0, 1):
        pltpu.sync_copy(zeros_hbm,
                        acc.at[pl.ds(sid * (_AR // _NS), _AR // _NS)])
        plsc.subcore_barrier()

        for seg in (0, 1):
            w = 2 * sid + seg
            nch = cnts[w, pl.ds(0, 16)][h]

            @pl.when(nch > 0)
            def _():
                pltpu.sync_copy(psrc.at[h, w], sball)
                _load_dball(lambda r: pdst.at[h, w, pl.ds(r * _CH, _CH)],
                            dball, nch, gsem,
                            psrc.at[0, 0, pl.ds(0, _CH)])
                _pipelined_chunks(nch, x_hbm, out_hbm.at[pl.ds(0, _CH)],
                                  sball, dball, rows_bf, rows_f, acc,
                                  gsem, ssem, convert=_convert)

        plsc.subcore_barrier()
        pltpu.sync_copy(
            acc.at[pl.ds(sid * (_HN // _NS), _HN // _NS)],
            out_hbm.at[pl.ds(h * _HN + sid * (_HN // _NS), _HN // _NS)])
        plsc.subcore_barrier()


def _agg_kernel(xlo, xhi, psrc, pdst, counts, zeros_a, out_lo, out_hi, sball,
                dball, cnts, rows_bf, rows_f, acc, gsem, ssem):
    cid = lax.axis_index("c")
    sid = lax.axis_index("s")

    @pl.when(cid == 0)
    def _():
        _agg_half_body(sid, xlo, psrc, pdst, counts, zeros_a, out_lo, sball,
                       dball, cnts, rows_bf, rows_f, acc, gsem, ssem)

    @pl.when(cid != 0)
    def _():
        _agg_half_body(sid, xhi, psrc, pdst, counts, zeros_a, out_hi, sball,
                       dball, cnts, rows_bf, rows_f, acc, gsem, ssem)


_aggregate = functools.partial(
    pl.kernel,
    _agg_kernel,
    out_type=[jax.ShapeDtypeStruct((_NP, _DP), jnp.float32),
              jax.ShapeDtypeStruct((_NP, _DP), jnp.float32)],
    mesh=_sc_mesh,
    scratch_types=[
        pltpu.VMEM((_CAP,), jnp.int32),
        pltpu.VMEM((_CAPC, _CH), jnp.int32),
        pltpu.VMEM((32, 16), jnp.int32),
        pltpu.VMEM((_NB, _CH, _DP), jnp.bfloat16),
        pltpu.VMEM((_NB, _CH, _DP), jnp.float32),
        pltpu.VMEM_SHARED((_AR, _DP), jnp.float32),
        pltpu.SemaphoreType.DMA((_NB,)),
        pltpu.SemaphoreType.DMA((_NB,)),
    ],
    compiler_params=pltpu.CompilerParams(use_tc_tiling_on_sc=False,
                                         needs_layout_passes=False),
)()


def _part_kernel(src, dst, psrc, pdst, counts, sin, din, bsrc0, bsrc1,
                 bdst0, bdst1, cbuf):
    cid = lax.axis_index("c")
    sid = lax.axis_index("s")
    w = sid * 2 + cid
    base = w * _CAP
    pltpu.sync_copy(src.at[pl.ds(base, _CAP)], sin)
    pltpu.sync_copy(dst.at[pl.ds(base, _CAP)], din)

    lanes = lax.iota(jnp.int32, 16)

    def _pre(i, _):
        spread = (i * 16 + lanes + w * 37) & 8191
        tloc = _HN + ((i * 16 + lanes) & 511)
        bsrc0[pl.ds(i * 16, 16)] = spread
        bsrc1[pl.ds(i * 16, 16)] = spread
        bdst0[pl.ds(i * 16, 16)] = tloc
        bdst1[pl.ds(i * 16, 16)] = tloc
        return 0
    lax.fori_loop(0, _CAP // 16, _pre, 0)

    def _scan(i, offs):
        o0, o1 = offs
        vs = sin[pl.ds(i * 16, 16)]
        vd = din[pl.ds(i * 16, 16)]
        m0 = vd < _HN
        m1 = ~m0
        ps0 = plsc.cumsum(m0.astype(jnp.int32))
        pos0 = o0 + ps0 - 1
        ps1 = plsc.cumsum(m1.astype(jnp.int32))
        pos1 = o1 + ps1 - 1
        plsc.store_scatter(bsrc0, [pos0], vs, mask=m0)
        plsc.store_scatter(bdst0, [pos0], vd, mask=m0)
        plsc.store_scatter(bsrc1, [pos1], vs, mask=m1)
        plsc.store_scatter(bdst1, [pos1], vd - _HN, mask=m1)
        c0 = ps0[15]
        return (o0 + c0, o1 + (16 - c0))
    o0, o1 = lax.fori_loop(0, _CAP // 16, _scan,
                           (jnp.int32(0), jnp.int32(0)))

    n0 = (o0 + (_CH - 1)) // _CH
    n1 = (o1 + (_CH - 1)) // _CH
    cbuf[pl.ds(0, 16)] = jnp.where(lanes == 0, n0,
                                   jnp.where(lanes == 1, n1, 0))
    pltpu.sync_copy(cbuf, counts.at[w])
    pltpu.sync_copy(bsrc0.at[pl.ds(0, _CAP)], psrc.at[0, w])
    pltpu.sync_copy(bsrc1.at[pl.ds(0, _CAP)], psrc.at[1, w])
    pltpu.sync_copy(bdst0.at[pl.ds(0, _CAP)], pdst.at[0, w])
    pltpu.sync_copy(bdst1.at[pl.ds(0, _CAP)], pdst.at[1, w])


_partition = functools.partial(
    pl.kernel,
    _part_kernel,
    out_type=[jax.ShapeDtypeStruct((2, 32, _CAP), jnp.int32),
              jax.ShapeDtypeStruct((2, 32, _CAP), jnp.int32),
              jax.ShapeDtypeStruct((32, 16), jnp.int32)],
    mesh=_sc_mesh,
    scratch_types=[
        pltpu.VMEM((_CAP,), jnp.int32),
        pltpu.VMEM((_CAP,), jnp.int32),
        pltpu.VMEM((_CAP + 16,), jnp.int32),
        pltpu.VMEM((_CAP + 16,), jnp.int32),
        pltpu.VMEM((_CAP + 16,), jnp.int32),
        pltpu.VMEM((_CAP + 16,), jnp.int32),
        pltpu.VMEM((16,), jnp.int32),
    ],
    compiler_params=pltpu.CompilerParams(use_tc_tiling_on_sc=False,
                                         needs_layout_passes=False),
)()


def _agg4_kernel(y4, src, dst, zeros_b, out_parts, sball, dball, rows, acc,
                 gsem, ssem):
    cid = lax.axis_index("c")
    sid = lax.axis_index("s")
    w = sid * 2 + cid

    start = w * _CPW * _CH
    pltpu.sync_copy(src.at[pl.ds(start, _CPW * _CH)], sball)
    _load_dball(lambda r: dst.at[pl.ds(start + r * _CH, _CH)], dball, _CPW,
                gsem, dst.at[pl.ds(0, _CH)])

    pltpu.sync_copy(zeros_b, acc.at[pl.ds(sid * _RPT, _RPT)])
    plsc.subcore_barrier()

    _pipelined_chunks(_CPW, y4, y4.at[pl.ds(0, _CH)], sball, dball, rows,
                      rows, acc, gsem, ssem)

    plsc.subcore_barrier()
    pltpu.sync_copy(acc.at[pl.ds(sid * _RPT, _RPT)],
                    out_parts.at[cid, pl.ds(sid * _RPT, _RPT)])


_aggregate4 = functools.partial(
    pl.kernel,
    _agg4_kernel,
    out_type=jax.ShapeDtypeStruct((2, _NP, _D4), jnp.float32),
    mesh=_sc_mesh,
    scratch_types=[
        pltpu.VMEM((_CPW * _CH,), jnp.int32),
        pltpu.VMEM((_CPW, _CH), jnp.int32),
        pltpu.VMEM((_NB, _CH, _D4), jnp.float32),
        pltpu.VMEM_SHARED((_NP, _D4), jnp.float32),
        pltpu.SemaphoreType.DMA((_NB,)),
        pltpu.SemaphoreType.DMA((_NB,)),
    ],
    compiler_params=pltpu.CompilerParams(use_tc_tiling_on_sc=False),
)()


def _dense_z(al_ref, ah_ref, wll, whl, wlh, whh, bl, bh):
    al = al_ref[...]
    ah = ah_ref[...]
    zl = (jnp.dot(al, wll[...], preferred_element_type=jnp.float32)
          + jnp.dot(ah, whl[...], preferred_element_type=jnp.float32)
          + bl[...])
    zh = (jnp.dot(al, wlh[...], preferred_element_type=jnp.float32)
          + jnp.dot(ah, whh[...], preferred_element_type=jnp.float32)
          + bh[...])
    return zl, zh


def _dense_unified_body(al_ref, ah_ref, wll, whl, wlh, whh, bl, bh,
                        w4l, w4h, ol_ref, oh_ref, y4_ref):
    zl, zh = _dense_z(al_ref, ah_ref, wll, whl, wlh, whh, bl, bh)
    m = jnp.maximum(zl.max(axis=-1, keepdims=True),
                    zh.max(axis=-1, keepdims=True))
    el = jnp.exp(zl - m)
    eh = jnp.exp(zh - m)
    s = el.sum(axis=-1, keepdims=True) + eh.sum(axis=-1, keepdims=True)
    ol_ref[...] = (el / s).astype(jnp.bfloat16)
    oh_ref[...] = (eh / s).astype(jnp.bfloat16)
    xl = jnp.maximum(zl, 0.0)
    xh = jnp.maximum(zh, 0.0)
    y4_ref[...] = (jnp.dot(xl, w4l[...], preferred_element_type=jnp.float32)
                   + jnp.dot(xh, w4h[...], preferred_element_type=jnp.float32))


def _final_body(p0_ref, p1_ref, b4_ref, out_ref):
    z = p0_ref[...] + p1_ref[...] + b4_ref[...]
    m = z.max(axis=-1, keepdims=True)
    s = jnp.exp(z - m).sum(axis=-1, keepdims=True)
    out_ref[...] = (z - m - jnp.log(s))[:, :3]


_row_spec = pl.BlockSpec((_R, _DP), lambda i: (i, 0))
_w_spec = pl.BlockSpec((_DP, _DP), lambda i: (0, 0))
_b_spec = pl.BlockSpec((1, _DP), lambda i: (0, 0))
_w4_spec = pl.BlockSpec((_DP, _D4), lambda i: (0, 0))
_y4_spec = pl.BlockSpec((_R, _D4), lambda i: (i, 0))
_half_out = jax.ShapeDtypeStruct((_NP, _DP), jnp.bfloat16)

_dense_unified = pl.pallas_call(
    _dense_unified_body,
    grid=(_GRID,),
    in_specs=[_row_spec, _row_spec, _w_spec, _w_spec, _w_spec, _w_spec,
              _b_spec, _b_spec, _w4_spec, _w4_spec],
    out_specs=[_row_spec, _row_spec, _y4_spec],
    out_shape=[_half_out, _half_out,
               jax.ShapeDtypeStruct((_NP, _D4), jnp.float32)],
)

_final = pl.pallas_call(
    _final_body,
    grid=(_GRID,),
    in_specs=[_y4_spec, _y4_spec,
              pl.BlockSpec((1, _D4), lambda i: (0, 0))],
    out_specs=pl.BlockSpec((_R, 3), lambda i: (i, 0)),
    out_shape=jax.ShapeDtypeStruct((_NP, 3), jnp.float32),
)


def _split_w(W):
    z = jnp.zeros((_DP, _DP), jnp.float32)
    wll = z.at[:_DH, :_DH].set(W[:_DH, :_DH])[_QIDX]
    whl = z.at[:_DH, :_DH].set(W[_DH:, :_DH])[_QIDX]
    wlh = z.at[:_DH, :_DH].set(W[:_DH, _DH:])[_QIDX]
    whh = z.at[:_DH, :_DH].set(W[_DH:, _DH:])[_QIDX]
    return wll, whl, wlh, whh


def _split_b(b, pad_val):
    bl = jnp.full((_DP,), pad_val, jnp.float32).at[:_DH].set(b[:_DH])
    bh = jnp.full((_DP,), pad_val, jnp.float32).at[:_DH].set(b[_DH:])
    return bl[None, :], bh[None, :]


def kernel(features, edge_index, W1, b1, W2, b2, W3, b3, W4, b4):
    edges = edge_index.astype(jnp.int32)
    npad = _EP - _E
    pad_src = (jnp.arange(npad, dtype=jnp.int32) * 131) % _N
    pad_dst = _N + (jnp.arange(npad, dtype=jnp.int32) % (_NP - _N))
    src = jnp.concatenate([edges[0], pad_src])
    dst = jnp.concatenate([edges[1], pad_dst])

    xlo = jnp.pad(features[:, :_DH],
                  ((0, _NP - _N), (0, _DP - _DH))).astype(jnp.bfloat16)
    xhi = jnp.pad(features[:, _DH:],
                  ((0, _NP - _N), (0, _DP - _DH))).astype(jnp.bfloat16)

    w4l = jnp.zeros((_DP, _D4), jnp.float32).at[:_DH, :3].set(W4[:_DH])
    w4h = jnp.zeros((_DP, _D4), jnp.float32).at[:_DH, :3].set(W4[_DH:])
    b4p = jnp.full((_D4,), _NEG, jnp.float32).at[:3].set(b4)[None, :]

    ws = [jnp.stack(p) for p in zip(_split_w(W1), _split_w(W2), _split_w(W3))]
    bs_l, bs_h = zip(_split_b(b1, _NEG), _split_b(b2, _NEG),
                     _split_b(b3, _NEG))
    xs = (*ws, jnp.stack(bs_l), jnp.stack(bs_h))

    zeros_a = jnp.zeros((_AR // _NS, _DP), jnp.float32)
    zeros_b = jnp.zeros((_RPT, _D4), jnp.float32)

    psrc, pdst, counts = _partition(src, dst)

    def _layer(carry, layer_ws):
        lo, hi = carry
        wll, whl, wlh, whh, bl, bh = layer_ws
        agg_lo, agg_hi = _aggregate(lo, hi, psrc, pdst, counts, zeros_a)
        nlo, nhi, y4 = _dense_unified(agg_lo, agg_hi, wll, whl, wlh, whh,
                                      bl, bh, w4l, w4h)
        return (nlo, nhi), y4

    _, y4s = lax.scan(_layer, (xlo, xhi), xs)
    parts = _aggregate4(y4s[2], src, dst, zeros_b)
    return _final(parts[0], parts[1], b4p)[:_N]


# speedup vs baseline: 1.6814x; 1.6814x over previous
import functools

import numpy as np

import jax
import jax.numpy as jnp
from jax import lax
from jax.experimental import pallas as pl
from jax.experimental.pallas import tpu as pltpu
from jax.experimental.pallas import tpu_sc as plsc

_N = 10000
_NP = 10240
_E = 160000
_D = 300
_DH = 150
_DP = 160
_CH = 80
_NS = 16
_RPT = _NP // _NS
_HN = _NP // 2
_AR = _HN + 512
_NCHUNK = 2048
_EP = _NCHUNK * _CH
_CPT = _NCHUNK // _NS
_CPW = _NCHUNK // (2 * _NS)
_CAP = _EP // 32
_CAPC = _CAP // _CH
_NB = 3
_D4 = 8
_NEG = -1e30

_R = 1024
_GRID = _NP // _R

_sc_mesh = plsc.VectorSubcoreMesh(core_axis_name="c", subcore_axis_name="s")

_QIDX = np.zeros(_DP, np.int32)
for _g in range(_DP // 32):
    for _i in range(16):
        _QIDX[32 * _g + _i] = 32 * _g + 2 * _i
        _QIDX[32 * _g + 16 + _i] = 32 * _g + 2 * _i + 1


def _pipelined_chunks(nch, x_hbm, dummy_f32, sball, dball, rows_in, rows_out,
                      acc, gsem, ssem, convert=None):
    nch = jnp.int32(nch)

    def _gwait(s):
        pltpu.make_async_copy(x_hbm.at[pl.ds(0, _CH)], rows_in.at[s],
                              gsem.at[s]).wait()

    def _swait(s):
        pltpu.make_async_copy(dummy_f32, rows_out.at[s], ssem.at[s]).wait()

    pltpu.async_copy(x_hbm.at[sball.at[pl.ds(0, _CH)]], rows_in.at[0],
                     gsem.at[0])

    def _chunk(g, _):
        s_g = lax.rem(g, _NB)
        nxt = g + 1

        @pl.when(nxt < nch)
        def _():
            pltpu.async_copy(x_hbm.at[sball.at[pl.ds(nxt * _CH, _CH)]],
                             rows_in.at[lax.rem(nxt, _NB)],
                             gsem.at[lax.rem(nxt, _NB)])

        _gwait(s_g)

        @pl.when(g >= _NB)
        def _():
            _swait(s_g)
        if convert is not None:
            convert(s_g)
        pltpu.async_copy(rows_out.at[s_g], acc.at[dball.at[g]], ssem.at[s_g],
                         add=True)
        return 0
    lax.fori_loop(0, nch, _chunk, 0)
    for j in range(_NB):
        @pl.when(j < nch)
        def _():
            _swait(j)


def _load_dball(row_slice_fn, dball, nch, sem, dummy):
    def _row(r, _):
        pltpu.async_copy(row_slice_fn(r), dball.at[r], sem.at[0])
        return 0
    lax.fori_loop(0, nch, _row, 0)

    def _drain(r, _):
        pltpu.make_async_copy(dummy, dball.at[0], sem.at[0]).wait()
        return 0
    lax.fori_loop(0, nch, _drain, 0)


def _agg_half_body(sid, x_hbm, psrc, pdst, counts, zeros_hbm, out_hbm, sball,
                   dball, cnts, rows_bf, rows_f, acc, gsem, ssem):
    pltpu.sync_copy(counts, cnts)

    def _convert(s):
        @plsc.parallel_loop(0, _CH, unroll=4)
        def _row(e):
            for g2 in range(_DP // 32):
                vb = plsc.bitcast(rows_bf[s, e, pl.ds(32 * g2, 32)],
                                  jnp.int32)
                rows_f[s, e, pl.ds(32 * g2, 16)] = plsc.bitcast(
                    vb << 16, jnp.float32)
                rows_f[s, e, pl.ds(32 * g2 + 16, 16)] = plsc.bitcast(
                    vb & jnp.int32(-65536), jnp.float32)

    for h in (0, 1):
        pltpu.sync_copy(zeros_hbm,
                        acc.at[pl.ds(sid * (_AR // _NS), _AR // _NS)])
        plsc.subcore_barrier()

        for seg in (0, 1):
            w = 2 * sid + seg
            nch = cnts[w, pl.ds(0, 16)][h]

            @pl.when(nch > 0)
            def _():
                pltpu.sync_copy(psrc.at[h, w], sball)
                _load_dball(lambda r: pdst.at[h, w, pl.ds(r * _CH, _CH)],
                            dball, nch, gsem,
                            psrc.at[0, 0, pl.ds(0, _CH)])
                _pipelined_chunks(nch, x_hbm, out_hbm.at[pl.ds(0, _CH)],
                                  sball, dball, rows_bf, rows_f, acc,
                                  gsem, ssem, convert=_convert)

        plsc.subcore_barrier()
        pltpu.sync_copy(
            acc.at[pl.ds(sid * (_HN // _NS), _HN // _NS)],
            out_hbm.at[pl.ds(h * _HN + sid * (_HN // _NS), _HN // _NS)])
        plsc.subcore_barrier()


def _agg_kernel(xlo, xhi, psrc, pdst, counts, zeros_a, out_lo, out_hi, sball,
                dball, cnts, rows_bf, rows_f, acc, gsem, ssem):
    cid = lax.axis_index("c")
    sid = lax.axis_index("s")

    @pl.when(cid == 0)
    def _():
        _agg_half_body(sid, xlo, psrc, pdst, counts, zeros_a, out_lo, sball,
                       dball, cnts, rows_bf, rows_f, acc, gsem, ssem)

    @pl.when(cid != 0)
    def _():
        _agg_half_body(sid, xhi, psrc, pdst, counts, zeros_a, out_hi, sball,
                       dball, cnts, rows_bf, rows_f, acc, gsem, ssem)


_aggregate = functools.partial(
    pl.kernel,
    _agg_kernel,
    out_type=[jax.ShapeDtypeStruct((_NP, _DP), jnp.float32),
              jax.ShapeDtypeStruct((_NP, _DP), jnp.float32)],
    mesh=_sc_mesh,
    scratch_types=[
        pltpu.VMEM((_CAP,), jnp.int32),
        pltpu.VMEM((_CAPC, _CH), jnp.int32),
        pltpu.VMEM((32, 16), jnp.int32),
        pltpu.VMEM((_NB, _CH, _DP), jnp.bfloat16),
        pltpu.VMEM((_NB, _CH, _DP), jnp.float32),
        pltpu.VMEM_SHARED((_AR, _DP), jnp.float32),
        pltpu.SemaphoreType.DMA((_NB,)),
        pltpu.SemaphoreType.DMA((_NB,)),
    ],
    compiler_params=pltpu.CompilerParams(use_tc_tiling_on_sc=False,
                                         needs_layout_passes=False),
)()


def _part_kernel(src, dst, psrc, pdst, counts, sin, din, bsrc0, bsrc1,
                 bdst0, bdst1, cbuf):
    cid = lax.axis_index("c")
    sid = lax.axis_index("s")
    w = sid * 2 + cid
    base = w * _CAP
    pltpu.sync_copy(src.at[pl.ds(base, _CAP)], sin)
    pltpu.sync_copy(dst.at[pl.ds(base, _CAP)], din)

    lanes = lax.iota(jnp.int32, 16)

    def _pre(i, _):
        spread = (i * 16 + lanes + w * 37) & 8191
        tloc = _HN + ((i * 16 + lanes) & 511)
        bsrc0[pl.ds(i * 16, 16)] = spread
        bsrc1[pl.ds(i * 16, 16)] = spread
        bdst0[pl.ds(i * 16, 16)] = tloc
        bdst1[pl.ds(i * 16, 16)] = tloc
        return 0
    lax.fori_loop(0, _CAP // 16, _pre, 0)

    def _scan(i, offs):
        o0, o1 = offs
        vs = sin[pl.ds(i * 16, 16)]
        vd = din[pl.ds(i * 16, 16)]
        m0 = vd < _HN
        m1 = ~m0
        ps0 = plsc.cumsum(m0.astype(jnp.int32))
        pos0 = o0 + ps0 - 1
        ps1 = plsc.cumsum(m1.astype(jnp.int32))
        pos1 = o1 + ps1 - 1
        plsc.store_scatter(bsrc0, [pos0], vs, mask=m0)
        plsc.store_scatter(bdst0, [pos0], vd, mask=m0)
        plsc.store_scatter(bsrc1, [pos1], vs, mask=m1)
        plsc.store_scatter(bdst1, [pos1], vd - _HN, mask=m1)
        c0 = ps0[15]
        return (o0 + c0, o1 + (16 - c0))
    o0, o1 = lax.fori_loop(0, _CAP // 16, _scan,
                           (jnp.int32(0), jnp.int32(0)))

    n0 = (o0 + (_CH - 1)) // _CH
    n1 = (o1 + (_CH - 1)) // _CH
    cbuf[pl.ds(0, 16)] = jnp.where(lanes == 0, n0,
                                   jnp.where(lanes == 1, n1, 0))
    pltpu.sync_copy(cbuf, counts.at[w])
    pltpu.sync_copy(bsrc0.at[pl.ds(0, _CAP)], psrc.at[0, w])
    pltpu.sync_copy(bsrc1.at[pl.ds(0, _CAP)], psrc.at[1, w])
    pltpu.sync_copy(bdst0.at[pl.ds(0, _CAP)], pdst.at[0, w])
    pltpu.sync_copy(bdst1.at[pl.ds(0, _CAP)], pdst.at[1, w])


_partition = functools.partial(
    pl.kernel,
    _part_kernel,
    out_type=[jax.ShapeDtypeStruct((2, 32, _CAP), jnp.int32),
              jax.ShapeDtypeStruct((2, 32, _CAP), jnp.int32),
              jax.ShapeDtypeStruct((32, 16), jnp.int32)],
    mesh=_sc_mesh,
    scratch_types=[
        pltpu.VMEM((_CAP,), jnp.int32),
        pltpu.VMEM((_CAP,), jnp.int32),
        pltpu.VMEM((_CAP + 16,), jnp.int32),
        pltpu.VMEM((_CAP + 16,), jnp.int32),
        pltpu.VMEM((_CAP + 16,), jnp.int32),
        pltpu.VMEM((_CAP + 16,), jnp.int32),
        pltpu.VMEM((16,), jnp.int32),
    ],
    compiler_params=pltpu.CompilerParams(use_tc_tiling_on_sc=False,
                                         needs_layout_passes=False),
)()


def _agg4_kernel(y4, src, dst, zeros_b, out_parts, sball, dball, rows, acc,
                 gsem, ssem):
    cid = lax.axis_index("c")
    sid = lax.axis_index("s")
    w = sid * 2 + cid

    start = w * _CPW * _CH
    pltpu.sync_copy(src.at[pl.ds(start, _CPW * _CH)], sball)
    _load_dball(lambda r: dst.at[pl.ds(start + r * _CH, _CH)], dball, _CPW,
                gsem, dst.at[pl.ds(0, _CH)])

    pltpu.sync_copy(zeros_b, acc.at[pl.ds(sid * _RPT, _RPT)])
    plsc.subcore_barrier()

    _pipelined_chunks(_CPW, y4, y4.at[pl.ds(0, _CH)], sball, dball, rows,
                      rows, acc, gsem, ssem)

    plsc.subcore_barrier()
    pltpu.sync_copy(acc.at[pl.ds(sid * _RPT, _RPT)],
                    out_parts.at[cid, pl.ds(sid * _RPT, _RPT)])


_aggregate4 = functools.partial(
    pl.kernel,
    _agg4_kernel,
    out_type=jax.ShapeDtypeStruct((2, _NP, _D4), jnp.float32),
    mesh=_sc_mesh,
    scratch_types=[
        pltpu.VMEM((_CPW * _CH,), jnp.int32),
        pltpu.VMEM((_CPW, _CH), jnp.int32),
        pltpu.VMEM((_NB, _CH, _D4), jnp.float32),
        pltpu.VMEM_SHARED((_NP, _D4), jnp.float32),
        pltpu.SemaphoreType.DMA((_NB,)),
        pltpu.SemaphoreType.DMA((_NB,)),
    ],
    compiler_params=pltpu.CompilerParams(use_tc_tiling_on_sc=False),
)()


def _dense_z(al_ref, ah_ref, wll, whl, wlh, whh, bl, bh):
    al = al_ref[...]
    ah = ah_ref[...]
    zl = (jnp.dot(al, wll[...], preferred_element_type=jnp.float32)
          + jnp.dot(ah, whl[...], preferred_element_type=jnp.float32)
          + bl[...])
    zh = (jnp.dot(al, wlh[...], preferred_element_type=jnp.float32)
          + jnp.dot(ah, whh[...], preferred_element_type=jnp.float32)
          + bh[...])
    return zl, zh


def _dense_unified_body(al_ref, ah_ref, wll, whl, wlh, whh, bl, bh,
                        w4l, w4h, ol_ref, oh_ref, y4_ref):
    zl, zh = _dense_z(al_ref, ah_ref, wll, whl, wlh, whh, bl, bh)
    m = jnp.maximum(zl.max(axis=-1, keepdims=True),
                    zh.max(axis=-1, keepdims=True))
    el = jnp.exp(zl - m)
    eh = jnp.exp(zh - m)
    s = el.sum(axis=-1, keepdims=True) + eh.sum(axis=-1, keepdims=True)
    ol_ref[...] = (el / s).astype(jnp.bfloat16)
    oh_ref[...] = (eh / s).astype(jnp.bfloat16)
    xl = jnp.maximum(zl, 0.0)
    xh = jnp.maximum(zh, 0.0)
    y4_ref[...] = (jnp.dot(xl, w4l[...], preferred_element_type=jnp.float32)
                   + jnp.dot(xh, w4h[...], preferred_element_type=jnp.float32))


def _final_body(p0_ref, p1_ref, b4_ref, out_ref):
    z = p0_ref[...] + p1_ref[...] + b4_ref[...]
    m = z.max(axis=-1, keepdims=True)
    s = jnp.exp(z - m).sum(axis=-1, keepdims=True)
    out_ref[...] = (z - m - jnp.log(s))[:, :3]


_row_spec = pl.BlockSpec((_R, _DP), lambda i: (i, 0))
_w_spec = pl.BlockSpec((_DP, _DP), lambda i: (0, 0))
_b_spec = pl.BlockSpec((1, _DP), lambda i: (0, 0))
_w4_spec = pl.BlockSpec((_DP, _D4), lambda i: (0, 0))
_y4_spec = pl.BlockSpec((_R, _D4), lambda i: (i, 0))
_half_out = jax.ShapeDtypeStruct((_NP, _DP), jnp.bfloat16)

_dense_unified = pl.pallas_call(
    _dense_unified_body,
    grid=(_GRID,),
    in_specs=[_row_spec, _row_spec, _w_spec, _w_spec, _w_spec, _w_spec,
              _b_spec, _b_spec, _w4_spec, _w4_spec],
    out_specs=[_row_spec, _row_spec, _y4_spec],
    out_shape=[_half_out, _half_out,
               jax.ShapeDtypeStruct((_NP, _D4), jnp.float32)],
)

_final = pl.pallas_call(
    _final_body,
    grid=(_GRID,),
    in_specs=[_y4_spec, _y4_spec,
              pl.BlockSpec((1, _D4), lambda i: (0, 0))],
    out_specs=pl.BlockSpec((_R, 3), lambda i: (i, 0)),
    out_shape=jax.ShapeDtypeStruct((_NP, 3), jnp.float32),
)


def _split_w(W):
    z = jnp.zeros((_DP, _DP), jnp.float32)
    wll = z.at[:_DH, :_DH].set(W[:_DH, :_DH])[_QIDX]
    whl = z.at[:_DH, :_DH].set(W[_DH:, :_DH])[_QIDX]
    wlh = z.at[:_DH, :_DH].set(W[:_DH, _DH:])[_QIDX]
    whh = z.at[:_DH, :_DH].set(W[_DH:, _DH:])[_QIDX]
    return wll, whl, wlh, whh


def _split_b(b, pad_val):
    bl = jnp.full((_DP,), pad_val, jnp.float32).at[:_DH].set(b[:_DH])
    bh = jnp.full((_DP,), pad_val, jnp.float32).at[:_DH].set(b[_DH:])
    return bl[None, :], bh[None, :]


def kernel(features, edge_index, W1, b1, W2, b2, W3, b3, W4, b4):
    edges = edge_index.astype(jnp.int32)
    npad = _EP - _E
    pad_src = (jnp.arange(npad, dtype=jnp.int32) * 131) % _N
    pad_dst = _N + (jnp.arange(npad, dtype=jnp.int32) % (_NP - _N))
    src = jnp.concatenate([edges[0], pad_src])
    dst = jnp.concatenate([edges[1], pad_dst])

    xlo = jnp.pad(features[:, :_DH],
                  ((0, _NP - _N), (0, _DP - _DH))).astype(jnp.bfloat16)
    xhi = jnp.pad(features[:, _DH:],
                  ((0, _NP - _N), (0, _DP - _DH))).astype(jnp.bfloat16)

    w4l = jnp.zeros((_DP, _D4), jnp.float32).at[:_DH, :3].set(W4[:_DH])
    w4h = jnp.zeros((_DP, _D4), jnp.float32).at[:_DH, :3].set(W4[_DH:])
    b4p = jnp.full((_D4,), _NEG, jnp.float32).at[:3].set(b4)[None, :]

    ws = [jnp.stack(p) for p in zip(_split_w(W1), _split_w(W2), _split_w(W3))]
    bs_l, bs_h = zip(_split_b(b1, _NEG), _split_b(b2, _NEG),
                     _split_b(b3, _NEG))
    xs = (*ws, jnp.stack(bs_l), jnp.stack(bs_h))

    zeros_a = jnp.zeros((_AR // _NS, _DP), jnp.float32)
    zeros_b = jnp.zeros((_RPT, _D4), jnp.float32)

    psrc, pdst, counts = _partition(src, dst)

    def _layer(carry, layer_ws):
        lo, hi = carry
        wll, whl, wlh, whh, bl, bh = layer_ws
        agg_lo, agg_hi = _aggregate(lo, hi, psrc, pdst, counts, zeros_a)
        nlo, nhi, y4 = _dense_unified(agg_lo, agg_hi, wll, whl, wlh, whh,
                                      bl, bh, w4l, w4h)
        return (nlo, nhi), y4

    _, y4s = lax.scan(_layer, (xlo, xhi), xs)
    parts = _aggregate4(y4s[2], src, dst, zeros_b)
    return _final(parts[0], parts[1], b4p)[:_N]


# speedup vs baseline: 1.8223x; 1.0838x over previous
import functools

import jax
import jax.numpy as jnp
from jax import lax
from jax.experimental import pallas as pl
from jax.experimental.pallas import tpu as pltpu
from jax.experimental.pallas import tpu_sc as plsc

_N = 10000
_NP = 10240
_E = 160000
_D = 300
_DH = 150
_DP = 160
_CH = 80
_NS = 16
_RPT = _NP // _NS
_HN = _NP // 2
_AR = _HN + 512
_NCHUNK = 2048
_EP = _NCHUNK * _CH
_CPT = _NCHUNK // _NS
_CPW = _NCHUNK // (2 * _NS)
_CAP = _EP // 32
_CAPC = _CAP // _CH
_NB = 3
_D4 = 8
_NEG = -1e30

_R = 1024
_GRID = _NP // _R

_sc_mesh = plsc.VectorSubcoreMesh(core_axis_name="c", subcore_axis_name="s")


def _pipelined_chunks(nch, x_hbm, sball, dball, rows, acc, gsem, ssem):
    nch = jnp.int32(nch)

    def _wait(sem_slot):
        pltpu.make_async_copy(x_hbm.at[pl.ds(0, _CH)], rows.at[sem_slot[0]],
                              sem_slot[1].at[sem_slot[0]]).wait()

    pltpu.async_copy(x_hbm.at[sball.at[pl.ds(0, _CH)]], rows.at[0],
                     gsem.at[0])

    def _chunk(g, _):
        s_g = lax.rem(g, _NB)
        nxt = g + 1

        @pl.when(nxt < nch)
        def _():
            s_n = lax.rem(nxt, _NB)

            @pl.when(g >= _NB - 1)
            def _():
                _wait((s_n, ssem))
            pltpu.async_copy(x_hbm.at[sball.at[pl.ds(nxt * _CH, _CH)]],
                             rows.at[s_n], gsem.at[s_n])

        _wait((s_g, gsem))
        pltpu.async_copy(rows.at[s_g], acc.at[dball.at[g]], ssem.at[s_g],
                         add=True)
        return 0
    lax.fori_loop(0, nch, _chunk, 0)
    for j in range(_NB):
        @pl.when(j < nch)
        def _():
            _wait((j, ssem))


def _load_dball(row_slice_fn, dball, nch, sem, dummy):
    def _row(r, _):
        pltpu.async_copy(row_slice_fn(r), dball.at[r], sem.at[0])
        return 0
    lax.fori_loop(0, nch, _row, 0)

    def _drain(r, _):
        pltpu.make_async_copy(dummy, dball.at[0], sem.at[0]).wait()
        return 0
    lax.fori_loop(0, nch, _drain, 0)


def _agg_half_body(sid, x_hbm, psrc, pdst, counts, zeros_hbm, out_hbm, sball,
                   dball, cnts, rows, acc, gsem, ssem):
    pltpu.sync_copy(counts, cnts)

    for h in (0, 1):
        pltpu.sync_copy(zeros_hbm,
                        acc.at[pl.ds(sid * (_AR // _NS), _AR // _NS)])
        plsc.subcore_barrier()

        for seg in (0, 1):
            w = 2 * sid + seg
            nch = cnts[w, pl.ds(0, 16)][h]

            @pl.when(nch > 0)
            def _():
                pltpu.sync_copy(psrc.at[h, w], sball)
                _load_dball(lambda r: pdst.at[h, w, pl.ds(r * _CH, _CH)],
                            dball, nch, gsem,
                            psrc.at[0, 0, pl.ds(0, _CH)])
                _pipelined_chunks(nch, x_hbm, sball, dball, rows, acc,
                                  gsem, ssem)

        plsc.subcore_barrier()
        pltpu.sync_copy(
            acc.at[pl.ds(sid * (_HN // _NS), _HN // _NS)],
            out_hbm.at[pl.ds(h * _HN + sid * (_HN // _NS), _HN // _NS)])
        plsc.subcore_barrier()


def _agg_kernel(xlo, xhi, psrc, pdst, counts, zeros_a, out_lo, out_hi, sball,
                dball, cnts, rows, acc, gsem, ssem):
    cid = lax.axis_index("c")
    sid = lax.axis_index("s")

    @pl.when(cid == 0)
    def _():
        _agg_half_body(sid, xlo, psrc, pdst, counts, zeros_a, out_lo, sball,
                       dball, cnts, rows, acc, gsem, ssem)

    @pl.when(cid != 0)
    def _():
        _agg_half_body(sid, xhi, psrc, pdst, counts, zeros_a, out_hi, sball,
                       dball, cnts, rows, acc, gsem, ssem)


_aggregate = functools.partial(
    pl.kernel,
    _agg_kernel,
    out_type=[jax.ShapeDtypeStruct((_NP, _DP), jnp.float32),
              jax.ShapeDtypeStruct((_NP, _DP), jnp.float32)],
    mesh=_sc_mesh,
    scratch_types=[
        pltpu.VMEM((_CAP,), jnp.int32),
        pltpu.VMEM((_CAPC, _CH), jnp.int32),
        pltpu.VMEM((32, 16), jnp.int32),
        pltpu.VMEM((_NB, _CH, _DP), jnp.float32),
        pltpu.VMEM_SHARED((_AR, _DP), jnp.float32),
        pltpu.SemaphoreType.DMA((_NB,)),
        pltpu.SemaphoreType.DMA((_NB,)),
    ],
    compiler_params=pltpu.CompilerParams(use_tc_tiling_on_sc=False),
)()


def _part_kernel(src, dst, psrc, pdst, counts, sin, din, bsrc0, bsrc1,
                 bdst0, bdst1, cbuf):
    cid = lax.axis_index("c")
    sid = lax.axis_index("s")
    w = sid * 2 + cid
    base = w * _CAP
    pltpu.sync_copy(src.at[pl.ds(base, _CAP)], sin)
    pltpu.sync_copy(dst.at[pl.ds(base, _CAP)], din)

    lanes = lax.iota(jnp.int32, 16)

    def _pre(i, _):
        spread = (i * 16 + lanes + w * 37) & 8191
        tloc = _HN + ((i * 16 + lanes) & 511)
        bsrc0[pl.ds(i * 16, 16)] = spread
        bsrc1[pl.ds(i * 16, 16)] = spread
        bdst0[pl.ds(i * 16, 16)] = tloc
        bdst1[pl.ds(i * 16, 16)] = tloc
        return 0
    lax.fori_loop(0, _CAP // 16, _pre, 0)

    def _scan(i, offs):
        o0, o1 = offs
        vs = sin[pl.ds(i * 16, 16)]
        vd = din[pl.ds(i * 16, 16)]
        m0 = vd < _HN
        m1 = ~m0
        ps0 = plsc.cumsum(m0.astype(jnp.int32))
        pos0 = o0 + ps0 - 1
        ps1 = plsc.cumsum(m1.astype(jnp.int32))
        pos1 = o1 + ps1 - 1
        plsc.store_scatter(bsrc0, [pos0], vs, mask=m0)
        plsc.store_scatter(bdst0, [pos0], vd, mask=m0)
        plsc.store_scatter(bsrc1, [pos1], vs, mask=m1)
        plsc.store_scatter(bdst1, [pos1], vd - _HN, mask=m1)
        c0 = ps0[15]
        return (o0 + c0, o1 + (16 - c0))
    o0, o1 = lax.fori_loop(0, _CAP // 16, _scan,
                           (jnp.int32(0), jnp.int32(0)))

    n0 = (o0 + (_CH - 1)) // _CH
    n1 = (o1 + (_CH - 1)) // _CH
    cbuf[pl.ds(0, 16)] = jnp.where(lanes == 0, n0,
                                   jnp.where(lanes == 1, n1, 0))
    pltpu.sync_copy(cbuf, counts.at[w])
    pltpu.sync_copy(bsrc0.at[pl.ds(0, _CAP)], psrc.at[0, w])
    pltpu.sync_copy(bsrc1.at[pl.ds(0, _CAP)], psrc.at[1, w])
    pltpu.sync_copy(bdst0.at[pl.ds(0, _CAP)], pdst.at[0, w])
    pltpu.sync_copy(bdst1.at[pl.ds(0, _CAP)], pdst.at[1, w])


_partition = functools.partial(
    pl.kernel,
    _part_kernel,
    out_type=[jax.ShapeDtypeStruct((2, 32, _CAP), jnp.int32),
              jax.ShapeDtypeStruct((2, 32, _CAP), jnp.int32),
              jax.ShapeDtypeStruct((32, 16), jnp.int32)],
    mesh=_sc_mesh,
    scratch_types=[
        pltpu.VMEM((_CAP,), jnp.int32),
        pltpu.VMEM((_CAP,), jnp.int32),
        pltpu.VMEM((_CAP + 16,), jnp.int32),
        pltpu.VMEM((_CAP + 16,), jnp.int32),
        pltpu.VMEM((_CAP + 16,), jnp.int32),
        pltpu.VMEM((_CAP + 16,), jnp.int32),
        pltpu.VMEM((16,), jnp.int32),
    ],
    compiler_params=pltpu.CompilerParams(use_tc_tiling_on_sc=False,
                                         needs_layout_passes=False),
)()


def _agg4_kernel(y4, src, dst, zeros_b, out_parts, sball, dball, rows, acc,
                 gsem, ssem):
    cid = lax.axis_index("c")
    sid = lax.axis_index("s")
    w = sid * 2 + cid

    start = w * _CPW * _CH
    pltpu.sync_copy(src.at[pl.ds(start, _CPW * _CH)], sball)
    _load_dball(lambda r: dst.at[pl.ds(start + r * _CH, _CH)], dball, _CPW,
                gsem, dst.at[pl.ds(0, _CH)])

    pltpu.sync_copy(zeros_b, acc.at[pl.ds(sid * _RPT, _RPT)])
    plsc.subcore_barrier()

    _pipelined_chunks(_CPW, y4, sball, dball, rows, acc, gsem, ssem)

    plsc.subcore_barrier()
    pltpu.sync_copy(acc.at[pl.ds(sid * _RPT, _RPT)],
                    out_parts.at[cid, pl.ds(sid * _RPT, _RPT)])


_aggregate4 = functools.partial(
    pl.kernel,
    _agg4_kernel,
    out_type=jax.ShapeDtypeStruct((2, _NP, _D4), jnp.float32),
    mesh=_sc_mesh,
    scratch_types=[
        pltpu.VMEM((_CPW * _CH,), jnp.int32),
        pltpu.VMEM((_CPW, _CH), jnp.int32),
        pltpu.VMEM((_NB, _CH, _D4), jnp.float32),
        pltpu.VMEM_SHARED((_NP, _D4), jnp.float32),
        pltpu.SemaphoreType.DMA((_NB,)),
        pltpu.SemaphoreType.DMA((_NB,)),
    ],
    compiler_params=pltpu.CompilerParams(use_tc_tiling_on_sc=False),
)()


def _dense_z(al_ref, ah_ref, wll, whl, wlh, whh, bl, bh):
    al = al_ref[...]
    ah = ah_ref[...]
    zl = (jnp.dot(al, wll[...], preferred_element_type=jnp.float32)
          + jnp.dot(ah, whl[...], preferred_element_type=jnp.float32)
          + bl[...])
    zh = (jnp.dot(al, wlh[...], preferred_element_type=jnp.float32)
          + jnp.dot(ah, whh[...], preferred_element_type=jnp.float32)
          + bh[...])
    return zl, zh


def _dense_unified_body(al_ref, ah_ref, wll, whl, wlh, whh, bl, bh,
                        w4l, w4h, ol_ref, oh_ref, y4_ref):
    zl, zh = _dense_z(al_ref, ah_ref, wll, whl, wlh, whh, bl, bh)
    m = jnp.maximum(zl.max(axis=-1, keepdims=True),
                    zh.max(axis=-1, keepdims=True))
    el = jnp.exp(zl - m)
    eh = jnp.exp(zh - m)
    s = el.sum(axis=-1, keepdims=True) + eh.sum(axis=-1, keepdims=True)
    ol_ref[...] = el / s
    oh_ref[...] = eh / s
    xl = jnp.maximum(zl, 0.0)
    xh = jnp.maximum(zh, 0.0)
    y4_ref[...] = (jnp.dot(xl, w4l[...], preferred_element_type=jnp.float32)
                   + jnp.dot(xh, w4h[...], preferred_element_type=jnp.float32))


def _final_body(p0_ref, p1_ref, b4_ref, out_ref):
    z = p0_ref[...] + p1_ref[...] + b4_ref[...]
    m = z.max(axis=-1, keepdims=True)
    s = jnp.exp(z - m).sum(axis=-1, keepdims=True)
    out_ref[...] = (z - m - jnp.log(s))[:, :3]


_row_spec = pl.BlockSpec((_R, _DP), lambda i: (i, 0))
_w_spec = pl.BlockSpec((_DP, _DP), lambda i: (0, 0))
_b_spec = pl.BlockSpec((1, _DP), lambda i: (0, 0))
_w4_spec = pl.BlockSpec((_DP, _D4), lambda i: (0, 0))
_y4_spec = pl.BlockSpec((_R, _D4), lambda i: (i, 0))
_half_out = jax.ShapeDtypeStruct((_NP, _DP), jnp.float32)

_dense_unified = pl.pallas_call(
    _dense_unified_body,
    grid=(_GRID,),
    in_specs=[_row_spec, _row_spec, _w_spec, _w_spec, _w_spec, _w_spec,
              _b_spec, _b_spec, _w4_spec, _w4_spec],
    out_specs=[_row_spec, _row_spec, _y4_spec],
    out_shape=[_half_out, _half_out,
               jax.ShapeDtypeStruct((_NP, _D4), jnp.float32)],
)

_final = pl.pallas_call(
    _final_body,
    grid=(_GRID,),
    in_specs=[_y4_spec, _y4_spec,
              pl.BlockSpec((1, _D4), lambda i: (0, 0))],
    out_specs=pl.BlockSpec((_R, 3), lambda i: (i, 0)),
    out_shape=jax.ShapeDtypeStruct((_NP, 3), jnp.float32),
)


def _split_w(W):
    z = jnp.zeros((_DP, _DP), jnp.float32)
    wll = z.at[:_DH, :_DH].set(W[:_DH, :_DH])
    whl = z.at[:_DH, :_DH].set(W[_DH:, :_DH])
    wlh = z.at[:_DH, :_DH].set(W[:_DH, _DH:])
    whh = z.at[:_DH, :_DH].set(W[_DH:, _DH:])
    return wll, whl, wlh, whh


def _split_b(b, pad_val):
    bl = jnp.full((_DP,), pad_val, jnp.float32).at[:_DH].set(b[:_DH])
    bh = jnp.full((_DP,), pad_val, jnp.float32).at[:_DH].set(b[_DH:])
    return bl[None, :], bh[None, :]


def kernel(features, edge_index, W1, b1, W2, b2, W3, b3, W4, b4):
    edges = edge_index.astype(jnp.int32)
    npad = _EP - _E
    pad_src = (jnp.arange(npad, dtype=jnp.int32) * 131) % _N
    pad_dst = _N + (jnp.arange(npad, dtype=jnp.int32) % (_NP - _N))
    src = jnp.concatenate([edges[0], pad_src])
    dst = jnp.concatenate([edges[1], pad_dst])

    xlo = jnp.pad(features[:, :_DH], ((0, _NP - _N), (0, _DP - _DH)))
    xhi = jnp.pad(features[:, _DH:], ((0, _NP - _N), (0, _DP - _DH)))

    w4l = jnp.zeros((_DP, _D4), jnp.float32).at[:_DH, :3].set(W4[:_DH])
    w4h = jnp.zeros((_DP, _D4), jnp.float32).at[:_DH, :3].set(W4[_DH:])
    b4p = jnp.full((_D4,), _NEG, jnp.float32).at[:3].set(b4)[None, :]

    ws = [jnp.stack(p) for p in zip(_split_w(W1), _split_w(W2), _split_w(W3))]
    bs_l, bs_h = zip(_split_b(b1, _NEG), _split_b(b2, _NEG),
                     _split_b(b3, _NEG))
    xs = (*ws, jnp.stack(bs_l), jnp.stack(bs_h))

    zeros_a = jnp.zeros((_AR // _NS, _DP), jnp.float32)
    zeros_b = jnp.zeros((_RPT, _D4), jnp.float32)

    psrc, pdst, counts = _partition(src, dst)

    def _layer(carry, layer_ws):
        lo, hi = carry
        wll, whl, wlh, whh, bl, bh = layer_ws
        agg_lo, agg_hi = _aggregate(lo, hi, psrc, pdst, counts, zeros_a)
        nlo, nhi, y4 = _dense_unified(agg_lo, agg_hi, wll, whl, wlh, whh,
                                      bl, bh, w4l, w4h)
        return (nlo, nhi), y4

    _, y4s = lax.scan(_layer, (xlo, xhi), xs)
    parts = _aggregate4(y4s[2], src, dst, zeros_b)
    return _final(parts[0], parts[1], b4p)[:_N]
